# initial kernel scaffold (unmeasured)
import jax
import jax.numpy as jnp
from jax import lax
from jax.experimental import pallas as pl
from jax.experimental.pallas import tpu as pltpu

N_DEV = 4
_GELU_C = 0.7978845608028654


def kernel(x, w_mat):
    m_per, k = x.shape
    n = w_mat.shape[1]
    n_per = n // N_DEV

    my = lax.axis_index("i")
    perm = (my + 1 + jnp.arange(N_DEV, dtype=jnp.int32)) % N_DEV

    def body(perm_ref, x_ref, w_ref, out_ref, y_send, send_sems, recv_sems):
        j = pl.program_id(0)
        me = lax.axis_index("i")

        y = jnp.dot(x_ref[...], w_ref[...], preferred_element_type=jnp.float32)
        y = 0.5 * y * (1.0 + jnp.tanh(_GELU_C * (y + 0.044715 * y * y * y)))
        y = y.astype(jnp.bfloat16)

        @pl.when(j < N_DEV - 1)
        def _send():
            y_send[j] = y
            rdma = pltpu.make_async_remote_copy(
                src_ref=y_send.at[j],
                dst_ref=out_ref.at[pl.ds(me * m_per, m_per), :],
                send_sem=send_sems.at[j],
                recv_sem=recv_sems.at[me],
                device_id=(perm_ref[j],),
                device_id_type=pl.DeviceIdType.MESH,
            )
            rdma.start()

        @pl.when(j == N_DEV - 1)
        def _finish():
            out_ref[pl.ds(me * m_per, m_per), :] = y
            for step in range(N_DEV - 1):
                src = (me - 1 - step) % N_DEV
                recv = pltpu.make_async_remote_copy(
                    src_ref=y_send.at[0],
                    dst_ref=out_ref.at[pl.ds(src * m_per, m_per), :],
                    send_sem=send_sems.at[0],
                    recv_sem=recv_sems.at[src],
                    device_id=(me,),
                    device_id_type=pl.DeviceIdType.MESH,
                )
                recv.wait_recv()
            for jj in range(N_DEV - 1):
                done = pltpu.make_async_remote_copy(
                    src_ref=y_send.at[jj],
                    dst_ref=out_ref.at[pl.ds(me * m_per, m_per), :],
                    send_sem=send_sems.at[jj],
                    recv_sem=recv_sems.at[me],
                    device_id=(me,),
                    device_id_type=pl.DeviceIdType.MESH,
                )
                done.wait_send()

    grid_spec = pltpu.PrefetchScalarGridSpec(
        num_scalar_prefetch=1,
        grid=(N_DEV,),
        in_specs=[
            pl.BlockSpec((m_per, k), lambda j, perm: (0, 0)),
            pl.BlockSpec((k, n_per), lambda j, perm: (0, perm[j])),
        ],
        out_specs=pl.BlockSpec((N_DEV * m_per, n_per), lambda j, perm: (0, 0)),
        scratch_shapes=[
            pltpu.VMEM((N_DEV - 1, m_per, n_per), jnp.bfloat16),
            pltpu.SemaphoreType.DMA((N_DEV - 1,)),
            pltpu.SemaphoreType.DMA((N_DEV,)),
        ],
    )
    return pl.pallas_call(
        body,
        grid_spec=grid_spec,
        out_shape=jax.ShapeDtypeStruct((N_DEV * m_per, n_per), jnp.bfloat16),
        compiler_params=pltpu.CompilerParams(
            dimension_semantics=("arbitrary",),
            collective_id=0,
        ),
    )(perm, x, w_mat)


# baseline (device time: 247712 ns/iter reference)
import jax
import jax.numpy as jnp
from jax import lax
from jax.experimental import pallas as pl
from jax.experimental.pallas import tpu as pltpu

N_DEV = 4
K_T = 16
_GELU_C = 0.7978845608028654


def kernel(x, w_mat):
    m_per, k = x.shape
    n = w_mat.shape[1]
    n_per = n // N_DEV
    k_t = k // K_T

    my = lax.axis_index("i")
    perm = (my + 1 + jnp.arange(N_DEV, dtype=jnp.int32)) % N_DEV

    def body(perm_ref, x_ref, w_ref, out_ref, acc, y_send,
             send_sems, recv_sems, copy_sem):
        j = pl.program_id(0)
        kk = pl.program_id(1)
        me = lax.axis_index("i")

        xa = x_ref[...].astype(jnp.bfloat16)
        wb = w_ref[...].astype(jnp.bfloat16)
        part = jnp.dot(xa, wb, preferred_element_type=jnp.float32)

        @pl.when(kk == 0)
        def _init():
            acc[...] = part

        @pl.when(kk > 0)
        def _accum():
            acc[...] = acc[...] + part

        @pl.when(kk == K_T - 1)
        def _block_done():
            slot = lax.rem(j, 2)

            @pl.when(j >= 2)
            def _():
                prev = pltpu.make_async_remote_copy(
                    src_ref=y_send.at[slot],
                    dst_ref=out_ref.at[pl.ds(me * m_per, m_per), :],
                    send_sem=send_sems.at[slot],
                    recv_sem=recv_sems.at[me],
                    device_id=(me,),
                    device_id_type=pl.DeviceIdType.MESH,
                )
                prev.wait_send()

            n_chunks = 8
            mc = m_per // n_chunks
            for c in range(n_chunks):
                a = acc[pl.ds(c * mc, mc), :]
                yc = 0.5 * a * (1.0 + jnp.tanh(_GELU_C * (a + 0.044715 * a * a * a)))
                y_send[slot, pl.ds(c * mc, mc), :] = yc.astype(jnp.bfloat16)

            @pl.when(j < N_DEV - 1)
            def _send():
                rdma = pltpu.make_async_remote_copy(
                    src_ref=y_send.at[slot],
                    dst_ref=out_ref.at[pl.ds(me * m_per, m_per), :],
                    send_sem=send_sems.at[slot],
                    recv_sem=recv_sems.at[me],
                    device_id=(perm_ref[j],),
                    device_id_type=pl.DeviceIdType.MESH,
                )
                rdma.start()

            @pl.when(j == N_DEV - 1)
            def _finish():
                local = pltpu.make_async_copy(
                    y_send.at[slot],
                    out_ref.at[pl.ds(me * m_per, m_per), :],
                    copy_sem,
                )
                local.start()
                for step in range(N_DEV - 1):
                    src = (me - 1 - step) % N_DEV
                    recv = pltpu.make_async_remote_copy(
                        src_ref=y_send.at[0],
                        dst_ref=out_ref.at[pl.ds(src * m_per, m_per), :],
                        send_sem=send_sems.at[0],
                        recv_sem=recv_sems.at[src],
                        device_id=(me,),
                        device_id_type=pl.DeviceIdType.MESH,
                    )
                    recv.wait_recv()
                last = pltpu.make_async_remote_copy(
                    src_ref=y_send.at[0],
                    dst_ref=out_ref.at[pl.ds(me * m_per, m_per), :],
                    send_sem=send_sems.at[0],
                    recv_sem=recv_sems.at[me],
                    device_id=(me,),
                    device_id_type=pl.DeviceIdType.MESH,
                )
                last.wait_send()
                local.wait()

    grid_spec = pltpu.PrefetchScalarGridSpec(
        num_scalar_prefetch=1,
        grid=(N_DEV, K_T),
        in_specs=[
            pl.BlockSpec((m_per, k_t), lambda j, kk, perm: (0, kk)),
            pl.BlockSpec((k_t, n_per), lambda j, kk, perm: (kk, perm[j])),
        ],
        out_specs=pl.BlockSpec(memory_space=pltpu.MemorySpace.HBM),
        scratch_shapes=[
            pltpu.VMEM((m_per, n_per), jnp.float32),
            pltpu.VMEM((2, m_per, n_per), jnp.bfloat16),
            pltpu.SemaphoreType.DMA((2,)),
            pltpu.SemaphoreType.DMA((N_DEV,)),
            pltpu.SemaphoreType.DMA,
        ],
    )
    return pl.pallas_call(
        body,
        grid_spec=grid_spec,
        out_shape=jax.ShapeDtypeStruct((N_DEV * m_per, n_per), jnp.bfloat16),
        compiler_params=pltpu.CompilerParams(
            dimension_semantics=("arbitrary", "arbitrary"),
            vmem_limit_bytes=60 * 1024 * 1024,
        ),
    )(perm, x, w_mat)
